# baseline (speedup 1.0000x reference)
import functools

import jax
import jax.numpy as jnp
from jax import lax
from jax.experimental import pallas as pl
from jax.experimental.pallas import tpu as pltpu
from jax.experimental.pallas import tpu_sc as plsc

_NUM_WORKERS = 32
_CHUNK = 1024
_GATHER = 128
_SLAB = 1 << 18
_VB = 2048


def _table_rowmajor(table_t):
    nblk = _SLAB // _VB

    def body(t0, t1, t2, t3, tout):
        for k, tk in enumerate((t0, t1, t2, t3)):
            tout[:, 32 * k:32 * (k + 1)] = jnp.transpose(tk[...])

    last_blk = table_t.shape[1] // _VB

    def in_spec(k):
        return pl.BlockSpec(
            (32, _VB),
            lambda b, k=k: (0, jnp.minimum(b + k * nblk, last_blk)),
        )

    return pl.pallas_call(
        body,
        grid=(nblk,),
        in_specs=[in_spec(0), in_spec(1), in_spec(2), in_spec(3)],
        out_specs=pl.BlockSpec((_VB, 128), lambda b: (b, 0)),
        out_shape=jax.ShapeDtypeStruct((_SLAB, 128), jnp.float32),
    )(table_t, table_t, table_t, table_t)


def _embed_gather(idx2, table, T, D):
    PW = T // _NUM_WORKERS
    NCH = PW // _CHUNK
    K = _CHUNK // _GATHER
    IDX_ROWS = PW // _GATHER

    mesh = plsc.VectorSubcoreMesh(core_axis_name="c", subcore_axis_name="s")

    @functools.partial(
        pl.kernel,
        mesh=mesh,
        out_type=jax.ShapeDtypeStruct((T, D), jnp.float32),
        compiler_params=pltpu.CompilerParams(use_tc_tiling_on_sc=False),
        scratch_types=[
            pltpu.VMEM((IDX_ROWS, _GATHER), jnp.int32),
            pltpu.VMEM((2 * _CHUNK, D), jnp.float32),
            pltpu.SemaphoreType.DMA,
            pltpu.SemaphoreType.DMA,
            pltpu.SemaphoreType.DMA,
            pltpu.SemaphoreType.DMA,
        ],
    )
    def run(x_hbm, tab_hbm, out_hbm,
            idx_v, rows_v, sem_g0, sem_g1, sem_o0, sem_o1):
        wid = lax.axis_index("s") * 2 + lax.axis_index("c")
        rowbase = wid * IDX_ROWS
        tokbase = wid * PW

        pltpu.sync_copy(x_hbm.at[pl.ds(rowbase, IDX_ROWS)], idx_v)

        sem_g = (sem_g0, sem_g1)
        sem_o = (sem_o0, sem_o1)
        gather_handles = [None, None]
        out_handles = [None, None]

        def fire_gathers(g):
            b = g % 2
            hs = []
            for j in range(K):
                src = tab_hbm.at[idx_v.at[g * K + j]]
                dst = rows_v.at[pl.ds(b * _CHUNK + j * _GATHER, _GATHER)]
                hs.append(pltpu.async_copy(src, dst, sem_g[b]))
            gather_handles[b] = hs

        def finish_chunk(g):
            b = g % 2
            for h in gather_handles[b]:
                h.wait()
            out_handles[b] = pltpu.async_copy(
                rows_v.at[pl.ds(b * _CHUNK, _CHUNK)],
                out_hbm.at[pl.ds(tokbase + g * _CHUNK, _CHUNK)],
                sem_o[b],
            )

        for g in range(NCH):
            b = g % 2
            if out_handles[b] is not None:
                out_handles[b].wait()
            fire_gathers(g)
            if g >= 1:
                finish_chunk(g - 1)
        finish_chunk(NCH - 1)
        for h in out_handles:
            h.wait()

    return run(idx2, table)


def _finalize(byrow, pos_col, B, L, D):
    def body(fin, pin, cout):
        cout[...] = jnp.transpose(fin[...]) + pin[...]

    return pl.pallas_call(
        body,
        grid=(B // 128,),
        in_specs=[
            pl.BlockSpec((128, L * D), lambda b: (b, 0)),
            pl.BlockSpec((L * D, 1), lambda b: (0, 0)),
        ],
        out_specs=pl.BlockSpec((L * D, 128), lambda b: (0, b)),
        out_shape=jax.ShapeDtypeStruct((L * D, B), jnp.float32),
    )(byrow, pos_col)


def kernel(x, table, pos):
    B, L = x.shape
    V, D = table.shape
    T = B * L

    xi = x.astype(jnp.int32)
    idxr = ((xi & (_SLAB - 1)) << 2) | (xi >> 18)
    idxp = idxr.reshape(B // 8, 8, L // 4, 4).transpose(0, 2, 1, 3)
    idx2 = idxp.reshape(T // _GATHER, _GATHER)
    pos_col = pos[0, :L, :].reshape(L * D, 1)

    tab_lin = _table_rowmajor(table.T).reshape(4 * _SLAB, D)
    out_sc = _embed_gather(idx2, tab_lin, T, D)
    byrow = (out_sc.reshape(B // 8, L // 4, 8, 4, D)
             .transpose(0, 2, 1, 3, 4).reshape(B, L * D))
    c2 = _finalize(byrow, pos_col, B, L, D)
    return c2.reshape(L, D, B).transpose(2, 0, 1)


# speedup vs baseline: 4.1854x; 4.1854x over previous
import functools

import jax
import jax.numpy as jnp
from jax import lax
from jax.experimental import pallas as pl
from jax.experimental.pallas import tpu as pltpu
from jax.experimental.pallas import tpu_sc as plsc

_NUM_WORKERS = 32
_CHUNK = 1024
_GATHER = 128
_SLAB = 1 << 18
_VB = 2048


def _table_rowmajor(table_t):
    nblk = _SLAB // _VB

    def body(t0, t1, t2, t3, tout):
        s = jnp.concatenate(
            [t0[...], t1[...], t2[...], t3[...]], axis=0)
        tout[...] = jnp.transpose(s)

    last_blk = table_t.shape[1] // _VB

    def in_spec(k):
        return pl.BlockSpec(
            (32, _VB),
            lambda b, k=k: (0, jnp.minimum(b + k * nblk, last_blk)),
        )

    return pl.pallas_call(
        body,
        grid=(nblk,),
        in_specs=[in_spec(0), in_spec(1), in_spec(2), in_spec(3)],
        out_specs=pl.BlockSpec((_VB, 128), lambda b: (b, 0)),
        out_shape=jax.ShapeDtypeStruct((_SLAB, 128), jnp.float32),
    )(table_t, table_t, table_t, table_t)


def _embed_gather(idx2, table, T, D):
    PW = T // _NUM_WORKERS
    NCH = PW // _CHUNK
    K = _CHUNK // _GATHER
    IDX_ROWS = PW // _GATHER

    mesh = plsc.VectorSubcoreMesh(core_axis_name="c", subcore_axis_name="s")

    @functools.partial(
        pl.kernel,
        mesh=mesh,
        out_type=jax.ShapeDtypeStruct((T, D), jnp.float32),
        compiler_params=pltpu.CompilerParams(use_tc_tiling_on_sc=False),
        scratch_types=[
            pltpu.VMEM((IDX_ROWS, _GATHER), jnp.int32),
            pltpu.VMEM((2 * _CHUNK, D), jnp.float32),
            pltpu.SemaphoreType.DMA,
            pltpu.SemaphoreType.DMA,
            pltpu.SemaphoreType.DMA,
            pltpu.SemaphoreType.DMA,
        ],
    )
    def run(x_hbm, tab_hbm, out_hbm,
            idx_v, rows_v, sem_g0, sem_g1, sem_o0, sem_o1):
        wid = lax.axis_index("s") * 2 + lax.axis_index("c")
        rowbase = wid * IDX_ROWS
        tokbase = wid * PW

        pltpu.sync_copy(x_hbm.at[pl.ds(rowbase, IDX_ROWS)], idx_v)

        sem_g = (sem_g0, sem_g1)
        sem_o = (sem_o0, sem_o1)
        gather_handles = [None, None]
        out_handles = [None, None]

        def fire_gathers(g):
            b = g % 2
            hs = []
            for j in range(K):
                src = tab_hbm.at[idx_v.at[g * K + j]]
                dst = rows_v.at[pl.ds(b * _CHUNK + j * _GATHER, _GATHER)]
                hs.append(pltpu.async_copy(src, dst, sem_g[b]))
            gather_handles[b] = hs

        def finish_chunk(g):
            b = g % 2
            for h in gather_handles[b]:
                h.wait()
            out_handles[b] = pltpu.async_copy(
                rows_v.at[pl.ds(b * _CHUNK, _CHUNK)],
                out_hbm.at[pl.ds(tokbase + g * _CHUNK, _CHUNK)],
                sem_o[b],
            )

        for g in range(NCH):
            b = g % 2
            if out_handles[b] is not None:
                out_handles[b].wait()
            fire_gathers(g)
            if g >= 1:
                finish_chunk(g - 1)
        finish_chunk(NCH - 1)
        for h in out_handles:
            h.wait()

    return run(idx2, table)


def _finalize(byrow, pos_col, B, L, D):
    def body(fin, pin, cout):
        cout[...] = jnp.transpose(fin[...]) + pin[...]

    return pl.pallas_call(
        body,
        grid=(B // 128,),
        in_specs=[
            pl.BlockSpec((128, L * D), lambda b: (b, 0)),
            pl.BlockSpec((L * D, 1), lambda b: (0, 0)),
        ],
        out_specs=pl.BlockSpec((L * D, 128), lambda b: (0, b)),
        out_shape=jax.ShapeDtypeStruct((L * D, B), jnp.float32),
    )(byrow, pos_col)


def kernel(x, table, pos):
    B, L = x.shape
    V, D = table.shape
    T = B * L

    xi = x.astype(jnp.int32)
    idxr = ((xi & (_SLAB - 1)) << 2) | (xi >> 18)
    idx2 = idxr.reshape(T // _GATHER, _GATHER)
    pos_col = pos[0, :L, :].reshape(L * D, 1)

    tab_lin = _table_rowmajor(table.T).reshape(4 * _SLAB, D)
    out_sc = _embed_gather(idx2, tab_lin, T, D)
    c2 = _finalize(out_sc.reshape(B, L * D), pos_col, B, L, D)
    return c2.reshape(L, D, B).transpose(2, 0, 1)


# speedup vs baseline: 4.6185x; 1.1035x over previous
import functools

import jax
import jax.numpy as jnp
from jax import lax
from jax.experimental import pallas as pl
from jax.experimental.pallas import tpu as pltpu
from jax.experimental.pallas import tpu_sc as plsc

_NUM_WORKERS = 32
_CHUNK = 1024
_GATHER = 128
_SLAB = 1 << 18
_VB = 4096


def _table_rowmajor(table_t):
    nblk = _SLAB // _VB

    def body(t0, t1, t2, t3, tout):
        s = jnp.concatenate(
            [t0[...], t1[...], t2[...], t3[...]], axis=0)
        tout[...] = jnp.transpose(s)

    last_blk = table_t.shape[1] // _VB

    def in_spec(k):
        return pl.BlockSpec(
            (32, _VB),
            lambda b, k=k: (0, jnp.minimum(b + k * nblk, last_blk)),
        )

    return pl.pallas_call(
        body,
        grid=(nblk,),
        in_specs=[in_spec(0), in_spec(1), in_spec(2), in_spec(3)],
        out_specs=pl.BlockSpec((_VB, 128), lambda b: (b, 0)),
        out_shape=jax.ShapeDtypeStruct((_SLAB, 128), jnp.float32),
    )(table_t, table_t, table_t, table_t)


def _embed_gather(idx2, table, T, D):
    PW = T // _NUM_WORKERS
    NCH = PW // _CHUNK
    K = _CHUNK // _GATHER
    IDX_ROWS = PW // _GATHER

    mesh = plsc.VectorSubcoreMesh(core_axis_name="c", subcore_axis_name="s")

    @functools.partial(
        pl.kernel,
        mesh=mesh,
        out_type=jax.ShapeDtypeStruct((T, D), jnp.float32),
        compiler_params=pltpu.CompilerParams(use_tc_tiling_on_sc=False),
        scratch_types=[
            pltpu.VMEM((IDX_ROWS, _GATHER), jnp.int32),
            pltpu.VMEM((2 * _CHUNK, D), jnp.float32),
            pltpu.SemaphoreType.DMA,
            pltpu.SemaphoreType.DMA,
            pltpu.SemaphoreType.DMA,
            pltpu.SemaphoreType.DMA,
        ],
    )
    def run(x_hbm, tab_hbm, out_hbm,
            idx_v, rows_v, sem_g0, sem_g1, sem_o0, sem_o1):
        wid = lax.axis_index("s") * 2 + lax.axis_index("c")
        rowbase = wid * IDX_ROWS
        tokbase = wid * PW

        pltpu.sync_copy(x_hbm.at[pl.ds(rowbase, IDX_ROWS)], idx_v)

        sem_g = (sem_g0, sem_g1)
        sem_o = (sem_o0, sem_o1)
        gather_handles = [None, None]
        out_handles = [None, None]

        def fire_gathers(g):
            b = g % 2
            hs = []
            for j in range(K):
                src = tab_hbm.at[idx_v.at[g * K + j]]
                dst = rows_v.at[pl.ds(b * _CHUNK + j * _GATHER, _GATHER)]
                hs.append(pltpu.async_copy(src, dst, sem_g[b]))
            gather_handles[b] = hs

        def finish_chunk(g):
            b = g % 2
            for h in gather_handles[b]:
                h.wait()
            out_handles[b] = pltpu.async_copy(
                rows_v.at[pl.ds(b * _CHUNK, _CHUNK)],
                out_hbm.at[pl.ds(tokbase + g * _CHUNK, _CHUNK)],
                sem_o[b],
            )

        for g in range(NCH):
            b = g % 2
            if out_handles[b] is not None:
                out_handles[b].wait()
            fire_gathers(g)
            if g >= 1:
                finish_chunk(g - 1)
        finish_chunk(NCH - 1)
        for h in out_handles:
            h.wait()

    return run(idx2, table)


def _finalize(byrow, pos_col, B, L, D):
    def body(fin, pin, cout):
        cout[...] = jnp.transpose(fin[...]) + pin[...]

    return pl.pallas_call(
        body,
        grid=(B // 128,),
        in_specs=[
            pl.BlockSpec((128, L * D), lambda b: (b, 0)),
            pl.BlockSpec((L * D, 1), lambda b: (0, 0)),
        ],
        out_specs=pl.BlockSpec((L * D, 128), lambda b: (0, b)),
        out_shape=jax.ShapeDtypeStruct((L * D, B), jnp.float32),
    )(byrow, pos_col)


def kernel(x, table, pos):
    B, L = x.shape
    V, D = table.shape
    T = B * L

    xi = x.astype(jnp.int32)
    idxr = ((xi & (_SLAB - 1)) << 2) | (xi >> 18)
    idx2 = idxr.reshape(T // _GATHER, _GATHER)
    pos_col = pos[0, :L, :].reshape(L * D, 1)

    tab_lin = _table_rowmajor(table.T).reshape(4 * _SLAB, D)
    out_sc = _embed_gather(idx2, tab_lin, T, D)
    c2 = _finalize(out_sc.reshape(B, L * D), pos_col, B, L, D)
    return c2.reshape(L, D, B).transpose(2, 0, 1)


# speedup vs baseline: 4.8015x; 1.0396x over previous
import functools

import jax
import jax.numpy as jnp
from jax import lax
from jax.experimental import pallas as pl
from jax.experimental.pallas import tpu as pltpu
from jax.experimental.pallas import tpu_sc as plsc

_NUM_WORKERS = 32
_CHUNK = 1024
_GATHER = 128
_SLAB = 1 << 18
_VB = 8192


def _table_rowmajor(table_t):
    nblk = _SLAB // _VB

    def body(t0, t1, t2, t3, tout):
        s = jnp.concatenate(
            [t0[...], t1[...], t2[...], t3[...]], axis=0)
        tout[...] = jnp.transpose(s)

    last_blk = table_t.shape[1] // _VB

    def in_spec(k):
        return pl.BlockSpec(
            (32, _VB),
            lambda b, k=k: (0, jnp.minimum(b + k * nblk, last_blk)),
        )

    return pl.pallas_call(
        body,
        grid=(nblk,),
        in_specs=[in_spec(0), in_spec(1), in_spec(2), in_spec(3)],
        out_specs=pl.BlockSpec((_VB, 128), lambda b: (b, 0)),
        out_shape=jax.ShapeDtypeStruct((_SLAB, 128), jnp.float32),
    )(table_t, table_t, table_t, table_t)


def _embed_gather(idx2, table, T, D):
    PW = T // _NUM_WORKERS
    NCH = PW // _CHUNK
    K = _CHUNK // _GATHER
    IDX_ROWS = PW // _GATHER

    mesh = plsc.VectorSubcoreMesh(core_axis_name="c", subcore_axis_name="s")

    @functools.partial(
        pl.kernel,
        mesh=mesh,
        out_type=jax.ShapeDtypeStruct((T, D), jnp.float32),
        compiler_params=pltpu.CompilerParams(use_tc_tiling_on_sc=False),
        scratch_types=[
            pltpu.VMEM((IDX_ROWS, _GATHER), jnp.int32),
            pltpu.VMEM((2 * _CHUNK, D), jnp.float32),
            pltpu.SemaphoreType.DMA,
            pltpu.SemaphoreType.DMA,
            pltpu.SemaphoreType.DMA,
            pltpu.SemaphoreType.DMA,
        ],
    )
    def run(x_hbm, tab_hbm, out_hbm,
            idx_v, rows_v, sem_g0, sem_g1, sem_o0, sem_o1):
        wid = lax.axis_index("s") * 2 + lax.axis_index("c")
        rowbase = wid * IDX_ROWS
        tokbase = wid * PW

        pltpu.sync_copy(x_hbm.at[pl.ds(rowbase, IDX_ROWS)], idx_v)

        sem_g = (sem_g0, sem_g1)
        sem_o = (sem_o0, sem_o1)
        gather_handles = [None, None]
        out_handles = [None, None]

        def fire_gathers(g):
            b = g % 2
            hs = []
            for j in range(K):
                src = tab_hbm.at[idx_v.at[g * K + j]]
                dst = rows_v.at[pl.ds(b * _CHUNK + j * _GATHER, _GATHER)]
                hs.append(pltpu.async_copy(src, dst, sem_g[b]))
            gather_handles[b] = hs

        def finish_chunk(g):
            b = g % 2
            for h in gather_handles[b]:
                h.wait()
            out_handles[b] = pltpu.async_copy(
                rows_v.at[pl.ds(b * _CHUNK, _CHUNK)],
                out_hbm.at[pl.ds(tokbase + g * _CHUNK, _CHUNK)],
                sem_o[b],
            )

        for g in range(NCH):
            b = g % 2
            if out_handles[b] is not None:
                out_handles[b].wait()
            fire_gathers(g)
            if g >= 1:
                finish_chunk(g - 1)
        finish_chunk(NCH - 1)
        for h in out_handles:
            h.wait()

    return run(idx2, table)


def _finalize(byrow, pos_col, B, L, D):
    def body(fin, pin, cout):
        cout[...] = jnp.transpose(fin[...]) + pin[...]

    return pl.pallas_call(
        body,
        grid=(B // 128,),
        in_specs=[
            pl.BlockSpec((128, L * D), lambda b: (b, 0)),
            pl.BlockSpec((L * D, 1), lambda b: (0, 0)),
        ],
        out_specs=pl.BlockSpec((L * D, 128), lambda b: (0, b)),
        out_shape=jax.ShapeDtypeStruct((L * D, B), jnp.float32),
    )(byrow, pos_col)


def kernel(x, table, pos):
    B, L = x.shape
    V, D = table.shape
    T = B * L

    xi = x.astype(jnp.int32)
    idxr = ((xi & (_SLAB - 1)) << 2) | (xi >> 18)
    idx2 = idxr.reshape(T // _GATHER, _GATHER)
    pos_col = pos[0, :L, :].reshape(L * D, 1)

    tab_lin = _table_rowmajor(table.T).reshape(4 * _SLAB, D)
    out_sc = _embed_gather(idx2, tab_lin, T, D)
    c2 = _finalize(out_sc.reshape(B, L * D), pos_col, B, L, D)
    return c2.reshape(L, D, B).transpose(2, 0, 1)
